# baseline (speedup 1.0000x reference)
import functools

import jax
import jax.numpy as jnp
from jax import lax
from jax.experimental import pallas as pl
from jax.experimental.pallas import tpu as pltpu
from jax.experimental.pallas import tpu_sc as plsc

_TARGET_WEIGHT = 0.5

_N = 16384 * 200
_NC = 2
_NS = 16
_NW = _NC * _NS
_PER_W = _N // _NW
_CHUNK = 12800
_NCH = _PER_W // _CHUNK
_GROUPS = _CHUNK // 64

_LN2 = 0.6931471805599453
_C5 = 0.030449
_C4 = -0.28382685
_C3 = 1.11609003
_C2 = -2.44002976
_C1 = 3.5140873
_C0 = -1.93675974 - 127.0 * _LN2


def _log_f32(t):
    bits = plsc.bitcast(t, jnp.int32)
    raw_e = jnp.right_shift(bits, 23)
    mant = plsc.bitcast((bits & 0x7FFFFF) | 0x3F800000, jnp.float32)
    ef = raw_e.astype(jnp.float32)
    poly = _C5 * mant + _C4
    poly = poly * mant + _C3
    poly = poly * mant + _C2
    poly = poly * mant + _C1
    poly = poly * mant + _C0
    logt = ef * _LN2 + poly
    return jnp.where(t > 0.0, logt, jnp.float32(-jnp.inf))


def _sc_body(p_hbm, m_hbm, out_hbm, pb0, pb1, mb0, mb1, vout,
             sp0, sp1, sm0, sm1):
    wid = lax.axis_index("s") * _NC + lax.axis_index("c")
    base = wid * _PER_W

    pbufs = (pb0, pb1)
    mbufs = (mb0, mb1)
    psems = (sp0, sp1)
    msems = (sm0, sm1)

    def start(i):
        b = i % 2
        off = pl.multiple_of(base + i * _CHUNK, _CHUNK)
        moff = pl.multiple_of((base + i * _CHUNK) // 4, _CHUNK // 4)
        cp = pltpu.make_async_copy(
            p_hbm.at[pl.ds(off, _CHUNK)], pbufs[b], psems[b])
        cm = pltpu.make_async_copy(
            m_hbm.at[pl.ds(moff, _CHUNK // 4)], mbufs[b], msems[b])
        cp.start()
        cm.start()
        return cp, cm

    iota = lax.iota(jnp.int32, 16)
    idxk = tuple(iota * 4 + k for k in range(4))
    zf = jnp.zeros((16,), jnp.float32)
    zi = jnp.zeros((16,), jnp.int32)

    def make_iter(pb, mb):
        def it(i, carry):
            acc_t, acc_n, cnt = carry
            off = i * 64
            w = mb[pl.ds(i * 16, 16)]
            nt, nn, nc = [], [], []
            for k in range(4):
                mk = (jnp.right_shift(w, 8 * k) & 1) if k else (w & 1)
                pred = mk == 1
                pv = plsc.load_gather(pb, [idxk[k] + off])
                t = jnp.where(pred, pv, 1.0 - pv)
                logt = _log_f32(t)
                nt.append(acc_t[k] + jnp.where(pred, logt, 0.0))
                nn.append(acc_n[k] + jnp.where(pred, 0.0, logt))
                nc.append(cnt[k] + mk)
            return tuple(nt), tuple(nn), tuple(nc)
        return it

    acc_t = (zf, zf, zf, zf)
    acc_n = (zf, zf, zf, zf)
    cnt = (zi, zi, zi, zi)

    inflight = {0: start(0)}
    for i in range(_NCH):
        cp, cm = inflight.pop(i)
        cp.wait()
        cm.wait()
        if i + 1 < _NCH:
            inflight[i + 1] = start(i + 1)
        b = i % 2
        acc_t, acc_n, cnt = lax.fori_loop(
            0, _GROUPS, make_iter(pbufs[b], mbufs[b]),
            (acc_t, acc_n, cnt), unroll=2)

    vout[0, :] = (acc_t[0] + acc_t[1]) + (acc_t[2] + acc_t[3])
    vout[1, :] = (acc_n[0] + acc_n[1]) + (acc_n[2] + acc_n[3])
    vout[2, :] = ((cnt[0] + cnt[1]) + (cnt[2] + cnt[3])).astype(jnp.float32)
    pltpu.sync_copy(vout, out_hbm.at[wid])


_sc_loss = functools.partial(
    pl.kernel,
    mesh=plsc.VectorSubcoreMesh(core_axis_name="c", subcore_axis_name="s"),
    out_type=jax.ShapeDtypeStruct((_NW, 3, 16), jnp.float32),
    compiler_params=pltpu.CompilerParams(needs_layout_passes=False),
    scratch_types=[
        pltpu.VMEM((_CHUNK,), jnp.float32),
        pltpu.VMEM((_CHUNK,), jnp.float32),
        pltpu.VMEM((_CHUNK // 4,), jnp.int32),
        pltpu.VMEM((_CHUNK // 4,), jnp.int32),
        pltpu.VMEM((3, 16), jnp.float32),
        pltpu.SemaphoreType.DMA,
        pltpu.SemaphoreType.DMA,
        pltpu.SemaphoreType.DMA,
        pltpu.SemaphoreType.DMA,
    ],
)(_sc_body)


def kernel(prob_pred, mask_gt):
    p = prob_pred.reshape(_N)
    m8 = mask_gt.reshape(_N).view(jnp.uint8)
    m32 = lax.bitcast_convert_type(m8.reshape(_N // 4, 4), jnp.int32)
    parts = _sc_loss(p, m32)
    sum_t = jnp.sum(parts[:, 0, :])
    sum_n = jnp.sum(parts[:, 1, :])
    n_t = jnp.sum(parts[:, 2, :])
    n_n = jnp.float32(_N) - n_t
    loss_t = -sum_t / n_t
    loss_n = -sum_n / n_n
    loss = _TARGET_WEIGHT * loss_t + (1.0 - _TARGET_WEIGHT) * loss_n
    return (loss, loss_t, loss_n)


# speedup vs baseline: 4.1067x; 4.1067x over previous
import functools

import jax
import jax.numpy as jnp
from jax import lax
from jax.experimental import pallas as pl
from jax.experimental.pallas import tpu as pltpu
from jax.experimental.pallas import tpu_sc as plsc

_TARGET_WEIGHT = 0.5

_N = 16384 * 200
_NC = 2
_NS = 16
_NW = _NC * _NS
_PER_W = _N // _NW
_CHUNK = 12800
_NCH = _PER_W // _CHUNK
_GROUPS = _CHUNK // 64

_LN2 = 0.6931471805599453
_C5 = 0.030449
_C4 = -0.28382685
_C3 = 1.11609003
_C2 = -2.44002976
_C1 = 3.5140873
_C0 = -1.93675974 - 127.0 * _LN2


def _log_f32(t):
    bits = plsc.bitcast(t, jnp.int32)
    raw_e = jnp.right_shift(bits, 23)
    mant = plsc.bitcast((bits & 0x7FFFFF) | 0x3F800000, jnp.float32)
    ef = raw_e.astype(jnp.float32)
    poly = _C5 * mant + _C4
    poly = poly * mant + _C3
    poly = poly * mant + _C2
    poly = poly * mant + _C1
    poly = poly * mant + _C0
    logt = ef * _LN2 + poly
    return jnp.where(t > 0.0, logt, jnp.float32(-jnp.inf))


def _sc_body(x_hbm, out_hbm, xb0, xb1, vout, sx0, sx1):
    wid = lax.axis_index("s") * _NC + lax.axis_index("c")
    base = wid * _PER_W

    xbufs = (xb0, xb1)
    xsems = (sx0, sx1)

    def start(i):
        b = i % 2
        off = pl.multiple_of(base + i * _CHUNK, _CHUNK)
        cp = pltpu.make_async_copy(
            x_hbm.at[pl.ds(off, _CHUNK)], xbufs[b], xsems[b])
        cp.start()
        return cp

    zf = jnp.zeros((16,), jnp.float32)

    def make_iter(xb):
        def it(i, carry):
            acc_t, acc_n, cnt = carry
            off = i * 64
            nt, nn, nc = [], [], []
            for k in range(4):
                x = xb[pl.ds(off + 16 * k, 16)]
                pred = x >= 0.0
                t = jnp.abs(x)
                logt = _log_f32(t)
                nt.append(acc_t[k] + jnp.where(pred, logt, 0.0))
                nn.append(acc_n[k] + jnp.where(pred, 0.0, logt))
                nc.append(cnt[k] + jnp.where(pred, 1.0, 0.0))
            return tuple(nt), tuple(nn), tuple(nc)
        return it

    acc_t = (zf, zf, zf, zf)
    acc_n = (zf, zf, zf, zf)
    cnt = (zf, zf, zf, zf)

    inflight = {0: start(0)}
    for i in range(_NCH):
        cp = inflight.pop(i)
        cp.wait()
        if i + 1 < _NCH:
            inflight[i + 1] = start(i + 1)
        acc_t, acc_n, cnt = lax.fori_loop(
            0, _GROUPS, make_iter(xbufs[i % 2]),
            (acc_t, acc_n, cnt), unroll=2)

    vout[0, :] = (acc_t[0] + acc_t[1]) + (acc_t[2] + acc_t[3])
    vout[1, :] = (acc_n[0] + acc_n[1]) + (acc_n[2] + acc_n[3])
    vout[2, :] = (cnt[0] + cnt[1]) + (cnt[2] + cnt[3])
    pltpu.sync_copy(vout, out_hbm.at[wid])


_sc_loss = functools.partial(
    pl.kernel,
    mesh=plsc.VectorSubcoreMesh(core_axis_name="c", subcore_axis_name="s"),
    out_type=jax.ShapeDtypeStruct((_NW, 3, 16), jnp.float32),
    compiler_params=pltpu.CompilerParams(needs_layout_passes=False),
    scratch_types=[
        pltpu.VMEM((_CHUNK,), jnp.float32),
        pltpu.VMEM((_CHUNK,), jnp.float32),
        pltpu.VMEM((3, 16), jnp.float32),
        pltpu.SemaphoreType.DMA,
        pltpu.SemaphoreType.DMA,
    ],
)(_sc_body)


def kernel(prob_pred, mask_gt):
    x = jnp.where(mask_gt, prob_pred, prob_pred - 1.0).reshape(_N)
    parts = _sc_loss(x)
    sum_t = jnp.sum(parts[:, 0, :])
    sum_n = jnp.sum(parts[:, 1, :])
    n_t = jnp.sum(parts[:, 2, :])
    n_n = jnp.float32(_N) - n_t
    loss_t = -sum_t / n_t
    loss_n = -sum_n / n_n
    loss = _TARGET_WEIGHT * loss_t + (1.0 - _TARGET_WEIGHT) * loss_n
    return (loss, loss_t, loss_n)


# speedup vs baseline: 6.7334x; 1.6396x over previous
import functools

import jax
import jax.numpy as jnp
from jax import lax
from jax.experimental import pallas as pl
from jax.experimental.pallas import tpu as pltpu
from jax.experimental.pallas import tpu_sc as plsc

_TARGET_WEIGHT = 0.5

_ROWS = 16384
_COLS = 200
_N = _ROWS * _COLS

_SC_ROWS = 1024
_TC_ROWS = _ROWS - _SC_ROWS
_TC_BLK = 1024
_TC_GRID = _TC_ROWS // _TC_BLK

_NC = 2
_NS = 16
_NW = _NC * _NS
_N_SC = _SC_ROWS * _COLS
_PER_W = _N_SC // _NW
_GROUPS = _PER_W // 64

_LN2 = 0.6931471805599453
_C5 = 0.030449
_C4 = -0.28382685
_C3 = 1.11609003
_C2 = -2.44002976
_C1 = 3.5140873
_C0 = -1.93675974 - 127.0 * _LN2


def _tc_body(p_ref, m_ref, out_ref):
    p = p_ref[...]
    mf = m_ref[...].astype(jnp.float32)
    mb = mf > 0.5
    t = jnp.where(mb, p, 1.0 - p)
    lt = jnp.log(t)
    out_ref[0, 0, 0] = jnp.sum(jnp.where(mb, lt, 0.0))
    out_ref[0, 0, 1] = jnp.sum(jnp.where(mb, 0.0, lt))
    out_ref[0, 0, 2] = jnp.sum(mf)


_tc_partials = pl.pallas_call(
    _tc_body,
    grid=(_TC_GRID,),
    in_specs=[
        pl.BlockSpec((_TC_BLK, _COLS), lambda i: (i, 0)),
        pl.BlockSpec((_TC_BLK, _COLS), lambda i: (i, 0)),
    ],
    out_specs=pl.BlockSpec((1, 1, 3), lambda i: (i, 0, 0),
                           memory_space=pltpu.SMEM),
    out_shape=jax.ShapeDtypeStruct((_TC_GRID, 1, 3), jnp.float32),
)


def _log_f32(t):
    bits = plsc.bitcast(t, jnp.int32)
    raw_e = jnp.right_shift(bits, 23)
    mant = plsc.bitcast((bits & 0x7FFFFF) | 0x3F800000, jnp.float32)
    ef = raw_e.astype(jnp.float32)
    poly = _C5 * mant + _C4
    poly = poly * mant + _C3
    poly = poly * mant + _C2
    poly = poly * mant + _C1
    poly = poly * mant + _C0
    logt = ef * _LN2 + poly
    return jnp.where(t > 0.0, logt, jnp.float32(-jnp.inf))


def _sc_body(x_hbm, out_hbm, xbuf, vout, sem):
    wid = lax.axis_index("s") * _NC + lax.axis_index("c")
    base = pl.multiple_of(wid * _PER_W, _PER_W)

    cp = pltpu.make_async_copy(x_hbm.at[pl.ds(base, _PER_W)], xbuf, sem)
    cp.start()

    zf = jnp.zeros((16,), jnp.float32)

    def it(i, carry):
        acc_t, acc_n, cnt = carry
        off = i * 64
        nt, nn, nc = [], [], []
        for k in range(4):
            x = xbuf[pl.ds(off + 16 * k, 16)]
            pred = x >= 0.0
            t = jnp.abs(x)
            logt = _log_f32(t)
            nt.append(acc_t[k] + jnp.where(pred, logt, 0.0))
            nn.append(acc_n[k] + jnp.where(pred, 0.0, logt))
            nc.append(cnt[k] + jnp.where(pred, 1.0, 0.0))
        return tuple(nt), tuple(nn), tuple(nc)

    cp.wait()
    acc_t, acc_n, cnt = lax.fori_loop(
        0, _GROUPS, it,
        ((zf,) * 4, (zf,) * 4, (zf,) * 4), unroll=2)

    vout[0, :] = (acc_t[0] + acc_t[1]) + (acc_t[2] + acc_t[3])
    vout[1, :] = (acc_n[0] + acc_n[1]) + (acc_n[2] + acc_n[3])
    vout[2, :] = (cnt[0] + cnt[1]) + (cnt[2] + cnt[3])
    pltpu.sync_copy(vout, out_hbm.at[wid])


_sc_partials = functools.partial(
    pl.kernel,
    mesh=plsc.VectorSubcoreMesh(core_axis_name="c", subcore_axis_name="s"),
    out_type=jax.ShapeDtypeStruct((_NW, 3, 16), jnp.float32),
    compiler_params=pltpu.CompilerParams(needs_layout_passes=False),
    scratch_types=[
        pltpu.VMEM((_PER_W,), jnp.float32),
        pltpu.VMEM((3, 16), jnp.float32),
        pltpu.SemaphoreType.DMA,
    ],
)(_sc_body)


def kernel(prob_pred, mask_gt):
    m8 = mask_gt.view(jnp.uint8)

    p_tail = prob_pred[_TC_ROWS:]
    m_tail = mask_gt[_TC_ROWS:]
    x = jnp.where(m_tail, p_tail, p_tail - 1.0).reshape(_N_SC)
    sc = _sc_partials(x)

    tc = _tc_partials(prob_pred, m8)

    sum_t = jnp.sum(tc[:, 0, 0]) + jnp.sum(sc[:, 0, :])
    sum_n = jnp.sum(tc[:, 0, 1]) + jnp.sum(sc[:, 1, :])
    n_t = jnp.sum(tc[:, 0, 2]) + jnp.sum(sc[:, 2, :])
    n_n = jnp.float32(_N) - n_t
    loss_t = -sum_t / n_t
    loss_n = -sum_n / n_n
    loss = _TARGET_WEIGHT * loss_t + (1.0 - _TARGET_WEIGHT) * loss_n
    return (loss, loss_t, loss_n)


# speedup vs baseline: 8.8533x; 1.3148x over previous
import functools

import jax
import jax.numpy as jnp
from jax import lax
from jax.experimental import pallas as pl
from jax.experimental.pallas import tpu as pltpu
from jax.experimental.pallas import tpu_sc as plsc

_TARGET_WEIGHT = 0.5

_ROWS = 16384
_COLS = 200
_N = _ROWS * _COLS

_SC_ROWS = 1024
_TC_ROWS = _ROWS - _SC_ROWS
_TC_BLK = 1024
_TC_GRID = _TC_ROWS // _TC_BLK

_NC = 2
_NS = 16
_NW = _NC * _NS
_N_SC = _SC_ROWS * _COLS
_PER_W = _N_SC // _NW
_GROUPS = _PER_W // 64

_LN2 = 0.6931471805599453
_C5 = 0.030449
_C4 = -0.28382685
_C3 = 1.11609003
_C2 = -2.44002976
_C1 = 3.5140873
_C0 = -1.93675974 - 127.0 * _LN2


def _tc_body(p_ref, m_ref, out_ref):
    p = p_ref[...]
    mb = m_ref[...]
    mf = mb.astype(jnp.float32)
    t = jnp.where(mb, p, 1.0 - p)
    lt = jnp.log(t)
    out_ref[0, 0, 0] = jnp.sum(jnp.where(mb, lt, 0.0))
    out_ref[0, 0, 1] = jnp.sum(jnp.where(mb, 0.0, lt))
    out_ref[0, 0, 2] = jnp.sum(mf)


_tc_partials = pl.pallas_call(
    _tc_body,
    grid=(_TC_GRID,),
    in_specs=[
        pl.BlockSpec((_COLS, _TC_BLK), lambda i: (0, i)),
        pl.BlockSpec((_COLS, _TC_BLK), lambda i: (0, i)),
    ],
    out_specs=pl.BlockSpec((1, 1, 3), lambda i: (i, 0, 0),
                           memory_space=pltpu.SMEM),
    out_shape=jax.ShapeDtypeStruct((_TC_GRID, 1, 3), jnp.float32),
)


def _log_f32(t):
    bits = plsc.bitcast(t, jnp.int32)
    raw_e = jnp.right_shift(bits, 23)
    mant = plsc.bitcast((bits & 0x7FFFFF) | 0x3F800000, jnp.float32)
    ef = raw_e.astype(jnp.float32)
    poly = _C5 * mant + _C4
    poly = poly * mant + _C3
    poly = poly * mant + _C2
    poly = poly * mant + _C1
    poly = poly * mant + _C0
    logt = ef * _LN2 + poly
    return jnp.where(t > 0.0, logt, jnp.float32(-jnp.inf))


def _sc_body(x_hbm, out_hbm, xbuf, vout, sem):
    wid = lax.axis_index("s") * _NC + lax.axis_index("c")
    base = pl.multiple_of(wid * _PER_W, _PER_W)

    cp = pltpu.make_async_copy(x_hbm.at[pl.ds(base, _PER_W)], xbuf, sem)
    cp.start()

    zf = jnp.zeros((16,), jnp.float32)

    def it(i, carry):
        acc_t, acc_n, cnt = carry
        off = i * 64
        nt, nn, nc = [], [], []
        for k in range(4):
            x = xbuf[pl.ds(off + 16 * k, 16)]
            pred = x >= 0.0
            t = jnp.abs(x)
            logt = _log_f32(t)
            nt.append(acc_t[k] + jnp.where(pred, logt, 0.0))
            nn.append(acc_n[k] + jnp.where(pred, 0.0, logt))
            nc.append(cnt[k] + jnp.where(pred, 1.0, 0.0))
        return tuple(nt), tuple(nn), tuple(nc)

    cp.wait()
    acc_t, acc_n, cnt = lax.fori_loop(
        0, _GROUPS, it,
        ((zf,) * 4, (zf,) * 4, (zf,) * 4), unroll=2)

    vout[0, :] = (acc_t[0] + acc_t[1]) + (acc_t[2] + acc_t[3])
    vout[1, :] = (acc_n[0] + acc_n[1]) + (acc_n[2] + acc_n[3])
    vout[2, :] = (cnt[0] + cnt[1]) + (cnt[2] + cnt[3])
    pltpu.sync_copy(vout, out_hbm.at[wid])


_sc_partials = functools.partial(
    pl.kernel,
    mesh=plsc.VectorSubcoreMesh(core_axis_name="c", subcore_axis_name="s"),
    out_type=jax.ShapeDtypeStruct((_NW, 3, 16), jnp.float32),
    compiler_params=pltpu.CompilerParams(needs_layout_passes=False),
    scratch_types=[
        pltpu.VMEM((_PER_W,), jnp.float32),
        pltpu.VMEM((3, 16), jnp.float32),
        pltpu.SemaphoreType.DMA,
    ],
)(_sc_body)


def kernel(prob_pred, mask_gt):
    p_tail = prob_pred[_TC_ROWS:]
    m_tail = mask_gt[_TC_ROWS:]
    x = jnp.where(m_tail, p_tail, p_tail - 1.0).reshape(_N_SC)
    sc = _sc_partials(x)

    tc = _tc_partials(prob_pred.T, mask_gt.T)

    sum_t = jnp.sum(tc[:, 0, 0]) + jnp.sum(sc[:, 0, :])
    sum_n = jnp.sum(tc[:, 0, 1]) + jnp.sum(sc[:, 1, :])
    n_t = jnp.sum(tc[:, 0, 2]) + jnp.sum(sc[:, 2, :])
    n_n = jnp.float32(_N) - n_t
    loss_t = -sum_t / n_t
    loss_n = -sum_n / n_n
    loss = _TARGET_WEIGHT * loss_t + (1.0 - _TARGET_WEIGHT) * loss_n
    return (loss, loss_t, loss_n)
